# baseline (speedup 1.0000x reference)
import jax
import jax.numpy as jnp
from jax.experimental import pallas as pl

_K = 8
_T = 1024
_SUB = 2


def _router_kernel(x_ref, w_ref, b_ref, idx_ref, val_ref):
    ts = _T // _SUB
    num_e = w_ref.shape[0]
    for st in range(_SUB):
        xs = x_ref[st * ts:(st + 1) * ts, :]
        logits = jax.lax.dot_general(
            w_ref[...], xs, (((1,), (1,)), ((), ())),
            preferred_element_type=jnp.float32)
        e = jnp.exp(logits + b_ref[...])
        s = jnp.sum(e, axis=0, keepdims=True)
        rev_iota = (jnp.int32(num_e - 1) - jax.lax.broadcasted_iota(
            jnp.int32, (num_e, 1), 0)).astype(jnp.float32)
        vals, ridx = [], []
        for it in range(_K):
            me = jnp.max(e, axis=0, keepdims=True)
            mi = jnp.max(jnp.where(e == me, rev_iota, jnp.float32(-1.0)),
                         axis=0, keepdims=True)
            vals.append(me)
            ridx.append(mi)
            if it != _K - 1:
                e = jnp.where(rev_iota == mi, jnp.float32(-1.0), e)
        valk = jnp.concatenate(vals, axis=0) / s
        ridxk = jnp.concatenate(ridx, axis=0)
        idx_ref[st * ts:(st + 1) * ts, :] = (
            jnp.int32(num_e - 1) - ridxk.T.astype(jnp.int32))
        val_ref[st * ts:(st + 1) * ts, :] = valk.T


def kernel(x, W, b):
    B, S, D = x.shape
    E = W.shape[0]
    N = B * S
    xf = x.reshape(N, D)
    b2 = b.reshape(E, 1)
    idx, val = pl.pallas_call(
        _router_kernel,
        grid=(N // _T,),
        in_specs=[
            pl.BlockSpec((_T, D), lambda i: (i, 0)),
            pl.BlockSpec((E, D), lambda i: (0, 0)),
            pl.BlockSpec((E, 1), lambda i: (0, 0)),
        ],
        out_specs=[
            pl.BlockSpec((_T, _K), lambda i: (i, 0)),
            pl.BlockSpec((_T, _K), lambda i: (i, 0)),
        ],
        out_shape=[
            jax.ShapeDtypeStruct((N, _K), jnp.int32),
            jax.ShapeDtypeStruct((N, _K), jnp.float32),
        ],
    )(xf, W, b2)
    return idx.reshape(B, S, _K), val.reshape(B, S, _K)


# speedup vs baseline: 1.0481x; 1.0481x over previous
import jax
import jax.numpy as jnp
from jax.experimental import pallas as pl

_K = 8
_T = 2048
_SUB = 2


def _router_kernel(x_ref, w_ref, b_ref, idx_ref, val_ref):
    ts = _T // _SUB
    num_e = w_ref.shape[0]
    for st in range(_SUB):
        xs = x_ref[st * ts:(st + 1) * ts, :]
        logits = jax.lax.dot_general(
            w_ref[...], xs, (((1,), (1,)), ((), ())),
            preferred_element_type=jnp.float32)
        e = jnp.exp(logits + b_ref[...])
        s = jnp.sum(e, axis=0, keepdims=True)
        rev_iota = (jnp.int32(num_e - 1) - jax.lax.broadcasted_iota(
            jnp.int32, (num_e, 1), 0)).astype(jnp.float32)
        vals, ridx = [], []
        for it in range(_K):
            me = jnp.max(e, axis=0, keepdims=True)
            mi = jnp.max(jnp.where(e == me, rev_iota, jnp.float32(-1.0)),
                         axis=0, keepdims=True)
            vals.append(me)
            ridx.append(mi)
            if it != _K - 1:
                e = jnp.where(rev_iota == mi, jnp.float32(-1.0), e)
        valk = jnp.concatenate(vals, axis=0) / s
        ridxk = jnp.concatenate(ridx, axis=0)
        idx_ref[st * ts:(st + 1) * ts, :] = (
            jnp.int32(num_e - 1) - ridxk.T.astype(jnp.int32))
        val_ref[st * ts:(st + 1) * ts, :] = valk.T


def kernel(x, W, b):
    B, S, D = x.shape
    E = W.shape[0]
    N = B * S
    xf = x.reshape(N, D)
    b2 = b.reshape(E, 1)
    idx, val = pl.pallas_call(
        _router_kernel,
        grid=(N // _T,),
        in_specs=[
            pl.BlockSpec((_T, D), lambda i: (i, 0)),
            pl.BlockSpec((E, D), lambda i: (0, 0)),
            pl.BlockSpec((E, 1), lambda i: (0, 0)),
        ],
        out_specs=[
            pl.BlockSpec((_T, _K), lambda i: (i, 0)),
            pl.BlockSpec((_T, _K), lambda i: (i, 0)),
        ],
        out_shape=[
            jax.ShapeDtypeStruct((N, _K), jnp.int32),
            jax.ShapeDtypeStruct((N, _K), jnp.float32),
        ],
    )(xf, W, b2)
    return idx.reshape(B, S, _K), val.reshape(B, S, _K)
